# baseline (speedup 1.0000x reference)
import functools

import jax
import jax.numpy as jnp
from jax import lax
from jax.experimental import pallas as pl
from jax.experimental.pallas import tpu as pltpu
from jax.experimental.pallas import tpu_sc as plsc

VOCAB = 1000000
EMBED_DIM = 32
BATCH = 4096
SEQ = 200

N = BATCH * SEQ
NC = 2
NS = 16
NW = NC * NS
PER_W = N // NW
CHUNK = 128
NCHUNK = PER_W // CHUNK
K = 8
NSUP = NCHUNK // K


def _emb_body(x_hbm, table_hbm, out_hbm, idx_v, rows_v, gsem, wsem):
    cid = lax.axis_index("c")
    sid = lax.axis_index("s")
    wid = sid * NC + cid
    base = wid * PER_W

    pltpu.sync_copy(x_hbm.at[wid], idx_v)

    def sup_body(s, carry):
        p = lax.rem(s, 2)
        sbase = base + s * (K * CHUNK)

        @pl.when(s >= 2)
        def _():
            for k in range(K):
                pltpu.make_async_copy(
                    rows_v.at[p, k],
                    out_hbm.at[pl.ds(sbase, CHUNK)],
                    wsem,
                ).wait()

        for k in range(K):
            pltpu.async_copy(
                table_hbm.at[idx_v.at[s * K + k]], rows_v.at[p, k], gsem
            )
        for k in range(K):
            pltpu.make_async_copy(
                table_hbm.at[idx_v.at[s * K + k]], rows_v.at[p, k], gsem
            ).wait()
        for k in range(K):
            pltpu.async_copy(
                rows_v.at[p, k],
                out_hbm.at[pl.ds(sbase + k * CHUNK, CHUNK)],
                wsem,
            )
        return carry

    lax.fori_loop(0, NSUP, sup_body, 0)

    for k in range(2 * K):
        pltpu.make_async_copy(
            rows_v.at[0, 0], out_hbm.at[pl.ds(base, CHUNK)], wsem
        ).wait()


def _prep_body(in_ref, out_ref):
    tpad = jnp.pad(in_ref[...], ((0, 96), (0, 0)))
    out_ref[...] = tpad.T


def _prep_table(table_t):
    return pl.pallas_call(
        _prep_body,
        grid=(1954,),
        in_specs=[pl.BlockSpec((32, 512), lambda j: (0, j))],
        out_specs=pl.BlockSpec((512, 128), lambda j: (j, 0)),
        out_shape=jax.ShapeDtypeStruct((VOCAB, 128), jnp.float32),
    )(table_t)


@jax.jit
def kernel(x, table):
    x_i32 = (x.astype(jnp.int32) * 4).reshape(NW, NCHUNK, CHUNK)
    tab4 = _prep_table(jnp.transpose(table)).reshape(4 * VOCAB, EMBED_DIM)
    mesh = plsc.VectorSubcoreMesh(core_axis_name="c", subcore_axis_name="s")
    f = functools.partial(
        pl.kernel,
        mesh=mesh,
        out_type=jax.ShapeDtypeStruct((N, EMBED_DIM), jnp.float32),
        scratch_types=[
            pltpu.VMEM((NCHUNK, CHUNK), jnp.int32),
            pltpu.VMEM((2, K, CHUNK, EMBED_DIM), jnp.float32),
            pltpu.SemaphoreType.DMA,
            pltpu.SemaphoreType.DMA,
        ],
        compiler_params=pltpu.CompilerParams(use_tc_tiling_on_sc=False),
    )(_emb_body)
    out = f(x_i32, tab4)
    return out.reshape(BATCH, SEQ, EMBED_DIM)


# speedup vs baseline: 1.6921x; 1.6921x over previous
import functools

import jax
import jax.numpy as jnp
from jax import lax
from jax.experimental import pallas as pl
from jax.experimental.pallas import tpu as pltpu
from jax.experimental.pallas import tpu_sc as plsc

VOCAB = 1000000
EMBED_DIM = 32
BATCH = 4096
SEQ = 200

N = BATCH * SEQ
NC = 2
NS = 16
NW = NC * NS
PER_W = N // NW
CHUNK = 128
NCHUNK = PER_W // CHUNK
K = 8
NSUP = NCHUNK // K


def _emb_body(x_hbm, table_hbm, out_hbm, idx_v, rows_v, gsem, wsem):
    cid = lax.axis_index("c")
    sid = lax.axis_index("s")
    wid = sid * NC + cid
    base = wid * PER_W

    pltpu.sync_copy(x_hbm.at[wid], idx_v)

    def sup_body(s, carry):
        p = lax.rem(s, 2)
        sbase = base + s * (K * CHUNK)

        @pl.when(s >= 2)
        def _():
            for k in range(K):
                pltpu.make_async_copy(
                    rows_v.at[p, k],
                    out_hbm.at[pl.ds(sbase, CHUNK)],
                    wsem,
                ).wait()

        for k in range(K):
            pltpu.async_copy(
                table_hbm.at[idx_v.at[s * K + k]], rows_v.at[p, k], gsem
            )
        for k in range(K):
            pltpu.make_async_copy(
                table_hbm.at[idx_v.at[s * K + k]], rows_v.at[p, k], gsem
            ).wait()
        for k in range(K):
            pltpu.async_copy(
                rows_v.at[p, k],
                out_hbm.at[pl.ds(sbase + k * CHUNK, CHUNK)],
                wsem,
            )
        return carry

    lax.fori_loop(0, NSUP, sup_body, 0)

    for k in range(2 * K):
        pltpu.make_async_copy(
            rows_v.at[0, 0], out_hbm.at[pl.ds(base, CHUNK)], wsem
        ).wait()


@jax.jit
def kernel(x, table):
    x_i32 = x.reshape(NW, NCHUNK, CHUNK).astype(jnp.int32)
    mesh = plsc.VectorSubcoreMesh(core_axis_name="c", subcore_axis_name="s")
    f = functools.partial(
        pl.kernel,
        mesh=mesh,
        out_type=jax.ShapeDtypeStruct((N, EMBED_DIM), jnp.float32),
        scratch_types=[
            pltpu.VMEM((NCHUNK, CHUNK), jnp.int32),
            pltpu.VMEM((2, K, CHUNK, EMBED_DIM), jnp.float32),
            pltpu.SemaphoreType.DMA,
            pltpu.SemaphoreType.DMA,
        ],
        compiler_params=pltpu.CompilerParams(use_tc_tiling_on_sc=False),
    )(_emb_body)
    out = f(x_i32, table)
    return out.reshape(BATCH, SEQ, EMBED_DIM)
